# Initial kernel scaffold; baseline (speedup 1.0000x reference)
import jax
import jax.numpy as jnp
from jax.experimental import pallas as pl


def kernel(grid_pos, img_fea, base_triangle2point, base_area_mask, base_triangle_mask, grid_size, output_pos):
    raise NotImplementedError("write your pallas kernel here")


# speedup vs baseline: 13.0732x; 13.0732x over previous
import functools

import jax
import jax.numpy as jnp
from jax import lax
from jax.experimental import pallas as pl
from jax.experimental.pallas import tpu as pltpu
from jax.experimental.pallas import tpu_sc as plsc

_B = 4
_G = 200
_G2 = 256
_N = 121
_C = 3
_P = 224 * 224
_PB = 1024
_NB = _P // _PB
_K = 8
_NEG_INF = float("-inf")


def _sc_gather(table, idx, total_rows, rows_per_worker):
    info = plsc.get_sparse_core_info()
    nc, ns = info.num_cores, info.num_subcores
    mesh = plsc.VectorSubcoreMesh(core_axis_name="c", subcore_axis_name="s")

    @functools.partial(
        pl.kernel,
        mesh=mesh,
        out_type=jax.ShapeDtypeStruct((total_rows, 128), jnp.float32),
        scratch_types=[
            pltpu.VMEM((rows_per_worker,), jnp.int32),
            pltpu.VMEM((rows_per_worker, 128), jnp.float32),
            pltpu.SemaphoreType.DMA,
        ],
    )
    def k(table_hbm, idx_hbm, out_hbm, idx_v, rows_v, sem):
        wid = lax.axis_index("s") * nc + lax.axis_index("c")
        base = wid * rows_per_worker
        pltpu.sync_copy(idx_hbm.at[pl.ds(base, rows_per_worker)], idx_v)
        pltpu.async_copy(table_hbm.at[idx_v], rows_v, sem).wait()
        pltpu.sync_copy(rows_v, out_hbm.at[pl.ds(base, rows_per_worker)])

    return k(table, idx)


def _pass_a_body(lat_ref, pos_ref, fea_ref, cond_ref, seg_ref):
    i = pl.program_id(1)
    lat = lat_ref[0]
    x0, y0 = lat[0:1, :], lat[1:2, :]
    x1, y1 = lat[2:3, :], lat[3:4, :]
    x2, y2 = lat[4:5, :], lat[5:6, :]
    xmin = jnp.minimum(jnp.minimum(x0, x1), x2)
    xmax = jnp.maximum(jnp.maximum(x0, x1), x2)
    ymin = jnp.minimum(jnp.minimum(y0, y1), y2)
    ymax = jnp.maximum(jnp.maximum(y0, y1), y2)

    qx = pos_ref[0, :, 0:1]
    qy = pos_ref[0, :, 1:2]
    g_iota = lax.broadcasted_iota(jnp.int32, (_PB, _G2), 1)
    valid = g_iota < _G
    inside = (qx >= xmin) & (qx <= xmax) & (qy >= ymin) & (qy <= ymax) & valid
    cand = jnp.where(inside, g_iota, _G2)
    cond = jnp.min(cand, axis=1, keepdims=True)
    cond = jnp.where(cond == _G2, 0, cond)
    cond_ref[0] = cond

    onehot = (g_iota == cond).astype(jnp.float32)
    fea = fea_ref[0]
    vals = jnp.concatenate(
        [fea, jnp.ones((_PB, 1), jnp.float32), jnp.zeros((_PB, 4), jnp.float32)],
        axis=1,
    )
    contrib = lax.dot_general(
        onehot, vals, (((0,), (0,)), ((), ())),
        preferred_element_type=jnp.float32,
        precision=lax.Precision.HIGHEST,
    )

    @pl.when(i == 0)
    def _():
        seg_ref[0] = contrib

    @pl.when(i > 0)
    def _():
        seg_ref[0] += contrib


def _pass_b_body(scal_ref, lat_ref, pos_ref, fea_ref, seg_ref,
                 recon_ref, stats_ref, av_ref):
    i = pl.program_id(1)
    neg_inv_sigma = scal_ref[0]
    lat = lat_ref[0]
    x0, y0 = lat[0:1, :], lat[1:2, :]
    x1, y1 = lat[2:3, :], lat[3:4, :]
    x2, y2 = lat[4:5, :], lat[5:6, :]
    cx = (x0 + x1 + x2) / 3.0
    cy = (y0 + y1 + y2) / 3.0

    seg = seg_ref[0]
    cnt = jnp.maximum(seg[:, 3:4], 1.0)
    gf = seg[:, 0:3] / cnt
    g2m = jnp.sum(gf * gf, axis=1, keepdims=True) * (1.0 / _C)
    gfe = jnp.concatenate([gf, g2m, jnp.zeros((_G2, 4), jnp.float32)], axis=1)

    qx = pos_ref[0, :, 0:1]
    qy = pos_ref[0, :, 1:2]
    dx = qx - cx
    dy = qy - cy
    d2 = dx * dx + dy * dy
    g_iota = lax.broadcasted_iota(jnp.int32, (_PB, _G2), 1)
    logits = jnp.where(g_iota < _G, d2 * neg_inv_sigma, _NEG_INF)

    l = logits
    m0 = jnp.max(l, axis=1, keepdims=True)
    wun = jnp.zeros((_PB, _G2), jnp.float32)
    ssum = jnp.zeros((_PB, 1), jnp.float32)
    for k in range(_K):
        m = m0 if k == 0 else jnp.max(l, axis=1, keepdims=True)
        is_m = l == m
        first = jnp.min(jnp.where(is_m, g_iota, _G2), axis=1, keepdims=True)
        sel = g_iota == first
        e = jnp.exp(m - m0)
        wun = wun + e * sel.astype(jnp.float32)
        ssum = ssum + e
        l = jnp.where(sel, _NEG_INF, l)
    w = wun / ssum

    r4 = lax.dot_general(
        w, gfe, (((1,), (0,)), ((), ())),
        preferred_element_type=jnp.float32,
        precision=lax.Precision.HIGHEST,
    )
    recon = r4[:, 0:3]
    recon_ref[0] = recon

    fea = fea_ref[0]
    fea2m = jnp.sum(fea * fea, axis=1, keepdims=True) * (1.0 / _C)
    dotfr = jnp.sum(fea * recon, axis=1, keepdims=True)
    varp = fea2m - (2.0 / _C) * dotfr + r4[:, 3:4]
    lossp = jnp.sum(jnp.abs(recon - fea), axis=1, keepdims=True)
    part = jnp.concatenate([varp, lossp, jnp.zeros((_PB, 6), jnp.float32)], axis=1)

    @pl.when(i == 0)
    def _():
        stats_ref[0] = part

    @pl.when(i > 0)
    def _():
        stats_ref[0] += part

    @pl.when(i == 0)
    def _():
        am = lat[6:7, :]
        ax, ay = 20.0 * x0, 20.0 * y0
        bx, by = 20.0 * x1, 20.0 * y1
        cx2, cy2 = 20.0 * x2, 20.0 * y2
        area1 = (ay + by) * (bx - ax) * 0.5
        area2 = (by + cy2) * (cx2 - bx) * 0.5
        area3 = (cy2 + ay) * (ax - cx2) * 0.5
        area = (area1 + area2 + area3) * am
        lane_valid = lax.broadcasted_iota(jnp.int32, (1, _G2), 1) < _G
        area = jnp.where(lane_valid, area, 0.0)
        mean = jnp.sum(area) / _G
        dev = jnp.where(lane_valid, area - mean, 0.0)
        av_ref[...] = jnp.broadcast_to(jnp.sum(dev * dev) / (_G - 1), (1, 1, 1))


def kernel(grid_pos, img_fea, base_triangle2point, base_area_mask,
           base_triangle_mask, grid_size, output_pos):
    del base_triangle_mask
    B, N = grid_pos.shape[0], grid_pos.shape[1]
    G = base_triangle2point.shape[1]

    table = jnp.pad(grid_pos.reshape(B * N, 2), ((0, 0), (0, 126)))
    idx = (base_triangle2point.reshape(B, G * 3)
           + (jnp.arange(B, dtype=jnp.int32) * N)[:, None]).reshape(-1)
    total = B * G * 3
    total_pad = 2560
    idx = jnp.pad(idx, (0, total_pad - total))
    rows = _sc_gather(table, idx, total_pad, total_pad // 32)
    lattice = rows[:total, :2].reshape(B, G, 3, 2)

    lat6 = lattice.reshape(B, G, 6)
    lat6 = jnp.pad(lat6, ((0, 0), (0, _G2 - G), (0, 0)))
    lat_t = jnp.transpose(lat6, (0, 2, 1))
    am_row = jnp.pad(base_area_mask, ((0, 0), (0, _G2 - G)))[:, None, :]
    lat_t = jnp.concatenate(
        [lat_t, am_row, jnp.zeros((B, 1, _G2), jnp.float32)], axis=1
    )

    pos = output_pos.reshape(_NB, _PB, 2)
    fea = img_fea.reshape(B, _P, _C)

    cond, seg = pl.pallas_call(
        _pass_a_body,
        grid=(B, _NB),
        in_specs=[
            pl.BlockSpec((1, 8, _G2), lambda b, i: (b, 0, 0)),
            pl.BlockSpec((1, _PB, 2), lambda b, i: (i, 0, 0)),
            pl.BlockSpec((1, _PB, _C), lambda b, i: (b, i, 0)),
        ],
        out_specs=[
            pl.BlockSpec((1, _PB, 1), lambda b, i: (b, i, 0)),
            pl.BlockSpec((1, _G2, 8), lambda b, i: (b, 0, 0)),
        ],
        out_shape=[
            jax.ShapeDtypeStruct((B, _P, 1), jnp.int32),
            jax.ShapeDtypeStruct((B, _G2, 8), jnp.float32),
        ],
        compiler_params=pltpu.CompilerParams(
            dimension_semantics=("arbitrary", "arbitrary"),
        ),
    )(lat_t, pos, fea)

    max_grid = jnp.maximum(grid_size[0] - 1, grid_size[1] - 1).astype(jnp.float32)
    neg_inv_sigma = jnp.reshape(-max_grid / 0.02, (1,))

    recon, stats, av = pl.pallas_call(
        _pass_b_body,
        grid=(B, _NB),
        in_specs=[
            pl.BlockSpec(memory_space=pltpu.SMEM),
            pl.BlockSpec((1, 8, _G2), lambda b, i: (b, 0, 0)),
            pl.BlockSpec((1, _PB, 2), lambda b, i: (i, 0, 0)),
            pl.BlockSpec((1, _PB, _C), lambda b, i: (b, i, 0)),
            pl.BlockSpec((1, _G2, 8), lambda b, i: (b, 0, 0)),
        ],
        out_specs=[
            pl.BlockSpec((1, _PB, _C), lambda b, i: (b, i, 0)),
            pl.BlockSpec((1, _PB, 8), lambda b, i: (b, 0, 0)),
            pl.BlockSpec((1, 1, 1), lambda b, i: (b, 0, 0)),
        ],
        out_shape=[
            jax.ShapeDtypeStruct((B, _P, _C), jnp.float32),
            jax.ShapeDtypeStruct((B, _PB, 8), jnp.float32),
            jax.ShapeDtypeStruct((B, 1, 1), jnp.float32),
        ],
        compiler_params=pltpu.CompilerParams(
            dimension_semantics=("arbitrary", "arbitrary"),
        ),
    )(neg_inv_sigma, lat_t, pos, fea, seg)

    condition = cond
    variance = jnp.sum(stats[:, :, 0], axis=1) / _P
    reconstruct_loss = jnp.sum(stats[:, :, 1], axis=1) / (_P * _C)
    area_variance = av[:, 0, 0]
    recon_img = recon.reshape(img_fea.shape)
    return (condition, lattice, variance, area_variance,
            reconstruct_loss, recon_img)


# speedup vs baseline: 28.6098x; 2.1884x over previous
import functools

import jax
import jax.numpy as jnp
from jax import lax
from jax.experimental import pallas as pl
from jax.experimental.pallas import tpu as pltpu
from jax.experimental.pallas import tpu_sc as plsc

_B = 4
_G = 200
_G2 = 256
_N = 121
_C = 3
_P = 224 * 224
_PB = 3584
_NB = _P // _PB
_K = 8


def _sc_gather(table, idx, total_rows, rows_per_worker):
    info = plsc.get_sparse_core_info()
    nc, ns = info.num_cores, info.num_subcores
    mesh = plsc.VectorSubcoreMesh(core_axis_name="c", subcore_axis_name="s")

    @functools.partial(
        pl.kernel,
        mesh=mesh,
        out_type=jax.ShapeDtypeStruct((total_rows, 128), jnp.float32),
        scratch_types=[
            pltpu.VMEM((rows_per_worker,), jnp.int32),
            pltpu.VMEM((rows_per_worker, 128), jnp.float32),
            pltpu.SemaphoreType.DMA,
        ],
    )
    def k(table_hbm, idx_hbm, out_hbm, idx_v, rows_v, sem):
        wid = lax.axis_index("s") * nc + lax.axis_index("c")
        base = wid * rows_per_worker
        pltpu.sync_copy(idx_hbm.at[pl.ds(base, rows_per_worker)], idx_v)
        pltpu.async_copy(table_hbm.at[idx_v], rows_v, sem).wait()
        pltpu.sync_copy(rows_v, out_hbm.at[pl.ds(base, rows_per_worker)])

    return k(table, idx)


def _pass_a_body(lat_ref, pos_ref, fea_ref, cond_ref, seg_ref):
    i = pl.program_id(1)
    lat = lat_ref[0]
    x0, y0 = lat[:, 0:1], lat[:, 1:2]
    x1, y1 = lat[:, 2:3], lat[:, 3:4]
    x2, y2 = lat[:, 4:5], lat[:, 5:6]
    xmin = jnp.minimum(jnp.minimum(x0, x1), x2)
    xmax = jnp.maximum(jnp.maximum(x0, x1), x2)
    ymin = jnp.minimum(jnp.minimum(y0, y1), y2)
    ymax = jnp.maximum(jnp.maximum(y0, y1), y2)

    qx = pos_ref[0:1, :]
    qy = pos_ref[1:2, :]
    inside = (qx >= xmin) & (qx <= xmax) & (qy >= ymin) & (qy <= ymax)
    g_f = lax.broadcasted_iota(jnp.int32, (_G2, 1), 0).astype(jnp.float32)
    candf = jnp.where(inside, g_f, float(_G2))
    condf = jnp.min(candf, axis=0, keepdims=True)
    condf = jnp.where(condf == float(_G2), 0.0, condf)
    cond_ref[0, 0] = condf.astype(jnp.int32)

    onehot = (g_f == condf).astype(jnp.bfloat16)
    fea = fea_ref[0]
    f_hi = fea.astype(jnp.bfloat16)
    f_lo = (fea - f_hi.astype(jnp.float32)).astype(jnp.bfloat16)
    vals16 = jnp.concatenate(
        [f_hi, jnp.ones((_PB, 1), jnp.bfloat16),
         jnp.zeros((_PB, 4), jnp.bfloat16),
         f_lo, jnp.zeros((_PB, 5), jnp.bfloat16)], axis=1)
    c16 = lax.dot_general(
        onehot, vals16, (((1,), (0,)), ((), ())),
        preferred_element_type=jnp.float32,
    )
    contrib = jnp.concatenate(
        [c16[:, 0:4] + c16[:, 8:12], jnp.zeros((_G2, 4), jnp.float32)], axis=1)

    @pl.when(i == 0)
    def _():
        seg_ref[0] = contrib

    @pl.when(i > 0)
    def _():
        seg_ref[0] += contrib


def _pass_b_body(scal_ref, lat_ref, pos_ref, fea_ref, seg_ref,
                 recon_ref, stats_ref, av_ref):
    i = pl.program_id(1)
    neg_inv_sigma = scal_ref[0]
    lat = lat_ref[0]
    x0, y0 = lat[:, 0:1], lat[:, 1:2]
    x1, y1 = lat[:, 2:3], lat[:, 3:4]
    x2, y2 = lat[:, 4:5], lat[:, 5:6]
    cx = (x0 + x1 + x2) / 3.0
    cy = (y0 + y1 + y2) / 3.0
    s_iota = lax.broadcasted_iota(jnp.int32, (_G2, 1), 0)
    cx = jnp.where(s_iota < _G, cx, 1e9)

    seg = seg_ref[0]
    cnt = jnp.maximum(seg[:, 3:4], 1.0)
    gf = seg[:, 0:3] / cnt
    g2m = jnp.sum(gf * gf, axis=1, keepdims=True) * (1.0 / _C)
    gfe = jnp.concatenate(
        [gf, g2m, jnp.ones((_G2, 1), jnp.float32),
         jnp.zeros((_G2, 3), jnp.float32)], axis=1)

    qx = pos_ref[0:1, :]
    qy = pos_ref[1:2, :]
    dx = qx - cx
    dy = qy - cy
    d2 = dx * dx + dy * dy
    logits = jnp.minimum(d2 * neg_inv_sigma, -1e-30)

    lb = lax.bitcast_convert_type(logits, jnp.int32)
    keyi = (~lb & jnp.int32(-256)) | (jnp.int32(255) - s_iota)
    key = lax.bitcast_convert_type(keyi, jnp.float32)
    m0k = jnp.max(key, axis=0, keepdims=True)
    for k in range(_K):
        mk = m0k if k == 0 else jnp.max(key, axis=0, keepdims=True)
        key = jnp.where(key == mk, 0.0, key)
    selm = key == 0.0
    lprime = lax.bitcast_convert_type(lb & jnp.int32(-256), jnp.float32)
    m0i = lax.bitcast_convert_type(m0k, jnp.int32)
    m0p = lax.bitcast_convert_type(~m0i & jnp.int32(-256), jnp.float32)
    wb = jnp.where(selm, jnp.exp(lprime - m0p), 0.0).astype(jnp.bfloat16)

    g_hi = gfe.astype(jnp.bfloat16)
    g_lo = (gfe - g_hi.astype(jnp.float32)).astype(jnp.bfloat16)
    gpack = jnp.concatenate([g_hi, g_lo], axis=1)
    r16 = lax.dot_general(
        wb, gpack, (((0,), (0,)), ((), ())),
        preferred_element_type=jnp.float32,
    )
    r4 = r16[:, 0:8] + r16[:, 8:16]
    r4 = r4 / r4[:, 4:5]
    recon = r4[:, 0:3]
    recon_ref[0] = recon

    fea = fea_ref[0]
    fea2m = jnp.sum(fea * fea, axis=1, keepdims=True) * (1.0 / _C)
    dotfr = jnp.sum(fea * recon, axis=1, keepdims=True)
    varp = fea2m - (2.0 / _C) * dotfr + r4[:, 3:4]
    lossp = jnp.sum(jnp.abs(recon - fea), axis=1, keepdims=True)
    part = jnp.concatenate([varp, lossp, jnp.zeros((_PB, 6), jnp.float32)], axis=1)

    @pl.when(i == 0)
    def _():
        stats_ref[0] = part

    @pl.when(i > 0)
    def _():
        stats_ref[0] += part

    @pl.when(i == 0)
    def _():
        am = lat[:, 6:7]
        ax, ay = 20.0 * x0, 20.0 * y0
        bx, by = 20.0 * x1, 20.0 * y1
        cx2, cy2 = 20.0 * x2, 20.0 * y2
        area1 = (ay + by) * (bx - ax) * 0.5
        area2 = (by + cy2) * (cx2 - bx) * 0.5
        area3 = (cy2 + ay) * (ax - cx2) * 0.5
        area = (area1 + area2 + area3) * am
        lane_valid = s_iota < _G
        area = jnp.where(lane_valid, area, 0.0)
        mean = jnp.sum(area) / _G
        dev = jnp.where(lane_valid, area - mean, 0.0)
        av_ref[...] = jnp.broadcast_to(jnp.sum(dev * dev) / (_G - 1), (1, 1, 1))


def kernel(grid_pos, img_fea, base_triangle2point, base_area_mask,
           base_triangle_mask, grid_size, output_pos):
    del base_triangle_mask
    B, N = grid_pos.shape[0], grid_pos.shape[1]
    G = base_triangle2point.shape[1]

    table = jnp.pad(grid_pos.reshape(B * N, 2), ((0, 0), (0, 126)))
    idx = (base_triangle2point.reshape(B, G * 3)
           + (jnp.arange(B, dtype=jnp.int32) * N)[:, None]).reshape(-1)
    total = B * G * 3
    total_pad = 2560
    idx = jnp.pad(idx, (0, total_pad - total))
    rows = _sc_gather(table, idx, total_pad, total_pad // 32)
    lattice = rows[:total, :2].reshape(B, G, 3, 2)

    lat6 = lattice.reshape(B, G, 6)
    lat6 = jnp.pad(lat6, ((0, 0), (0, _G2 - G), (0, 0)))
    am_col = jnp.pad(base_area_mask, ((0, 0), (0, _G2 - G)))[:, :, None]
    lat8 = jnp.concatenate(
        [lat6, am_col, jnp.zeros((B, _G2, 1), jnp.float32)], axis=2)

    pos_t = output_pos.reshape(_P, 2).T
    fea = img_fea.reshape(B, _P, _C)

    cond4, seg = pl.pallas_call(
        _pass_a_body,
        grid=(B, _NB),
        in_specs=[
            pl.BlockSpec((1, _G2, 8), lambda b, i: (b, 0, 0)),
            pl.BlockSpec((2, _PB), lambda b, i: (0, i)),
            pl.BlockSpec((1, _PB, _C), lambda b, i: (b, i, 0)),
        ],
        out_specs=[
            pl.BlockSpec((1, 1, 1, _PB), lambda b, i: (b, i, 0, 0)),
            pl.BlockSpec((1, _G2, 8), lambda b, i: (b, 0, 0)),
        ],
        out_shape=[
            jax.ShapeDtypeStruct((B, _NB, 1, _PB), jnp.int32),
            jax.ShapeDtypeStruct((B, _G2, 8), jnp.float32),
        ],
        compiler_params=pltpu.CompilerParams(
            dimension_semantics=("arbitrary", "arbitrary"),
        ),
    )(lat8, pos_t, fea)

    max_grid = jnp.maximum(grid_size[0] - 1, grid_size[1] - 1).astype(jnp.float32)
    neg_inv_sigma = jnp.reshape(-max_grid / 0.02, (1,))

    recon, stats, av = pl.pallas_call(
        _pass_b_body,
        grid=(B, _NB),
        in_specs=[
            pl.BlockSpec(memory_space=pltpu.SMEM),
            pl.BlockSpec((1, _G2, 8), lambda b, i: (b, 0, 0)),
            pl.BlockSpec((2, _PB), lambda b, i: (0, i)),
            pl.BlockSpec((1, _PB, _C), lambda b, i: (b, i, 0)),
            pl.BlockSpec((1, _G2, 8), lambda b, i: (b, 0, 0)),
        ],
        out_specs=[
            pl.BlockSpec((1, _PB, _C), lambda b, i: (b, i, 0)),
            pl.BlockSpec((1, _PB, 8), lambda b, i: (b, 0, 0)),
            pl.BlockSpec((1, 1, 1), lambda b, i: (b, 0, 0)),
        ],
        out_shape=[
            jax.ShapeDtypeStruct((B, _P, _C), jnp.float32),
            jax.ShapeDtypeStruct((B, _PB, 8), jnp.float32),
            jax.ShapeDtypeStruct((B, 1, 1), jnp.float32),
        ],
        compiler_params=pltpu.CompilerParams(
            dimension_semantics=("arbitrary", "arbitrary"),
        ),
    )(neg_inv_sigma, lat8, pos_t, fea, seg)

    condition = cond4.reshape(B, _P, 1)
    variance = jnp.sum(stats[:, :, 0], axis=1) / _P
    reconstruct_loss = jnp.sum(stats[:, :, 1], axis=1) / (_P * _C)
    area_variance = av[:, 0, 0]
    recon_img = recon.reshape(img_fea.shape)
    return (condition, lattice, variance, area_variance,
            reconstruct_loss, recon_img)
